# Initial kernel scaffold; baseline (speedup 1.0000x reference)
import jax
import jax.numpy as jnp
from jax.experimental import pallas as pl


def kernel(forest, adjacency, node_order, edge_order, W_iou_w, W_iou_b, U_iou_w, W_c_w, W_c_b, W_f_w, W_f_b, U_f_w):
    raise NotImplementedError("write your pallas kernel here")


# speedup vs baseline: 5.6488x; 5.6488x over previous
import jax
import jax.numpy as jnp
from jax.experimental import pallas as pl
from jax.experimental.pallas import tpu as pltpu

N = 10000
E = 320000
F = 128
BF = 3
MAX_IT = 16
C = 16384
NCHUNK = -(-E // C)
EPAD = NCHUNK * C
TILE = 1000
NT = N // TILE


def _body(offs_ref, pe_ref, forest_ref, nord_ref,
          wf_t_ref, wf_b_ref, wiou_t_ref, wiou_b_ref,
          uiou_t_ref, uf_t_ref, wc_t_ref, wc_b_ref,
          h_out_ref,
          c_s, g_s, fp_s, fh_s, fc_s):
    f32 = jnp.float32
    t = pl.program_id(0)
    j = pl.program_id(1)

    def mm(a, b):
        return jnp.dot(a, b, preferred_element_type=f32)

    @pl.when((t == 0) & (j == 0))
    def _init():
        for r in range(NT):
            sl = pl.ds(r * TILE, TILE)
            fp_s[sl, :] = mm(forest_ref[sl, :], wf_t_ref[...]) + wf_b_ref[...]
            h_out_ref[sl, :] = jnp.zeros((TILE, F), f32)
            c_s[sl, :] = jnp.zeros((TILE, F), f32)

    @pl.when(j == 0)
    def _start_iteration():
        for r in range(NT):
            sl = pl.ds(r * TILE, TILE)
            for s in range(BF):
                fh_s[s, sl, :] = jnp.zeros((TILE, F), f32)
                fc_s[s, sl, :] = jnp.zeros((TILE, F), f32)
            g_s[sl, :] = mm(h_out_ref[sl, :], uf_t_ref[...])

    base = j * C
    lo = jnp.maximum(offs_ref[t], base)
    hi = jnp.minimum(offs_ref[t + 1], base + C)

    def edge_body(e, _):
        packed = pe_ref[e - base]
        child = packed & 0x3FFF
        slot = (packed >> 14) & 0x3
        parent = (packed >> 16) & 0x3FFF
        hrow = h_out_ref[pl.ds(child, 1), :]
        crow = c_s[pl.ds(child, 1), :]
        grow = g_s[pl.ds(child, 1), :]
        prow = fp_s[pl.ds(parent, 1), :]
        f = jax.nn.sigmoid(prow + grow)
        fcrow = f * crow
        cur_h = fh_s[pl.ds(slot, 1), pl.ds(parent, 1), :]
        fh_s[pl.ds(slot, 1), pl.ds(parent, 1), :] = cur_h + hrow.reshape(1, 1, F)
        cur_c = fc_s[pl.ds(slot, 1), pl.ds(parent, 1), :]
        fc_s[pl.ds(slot, 1), pl.ds(parent, 1), :] = cur_c + fcrow.reshape(1, 1, F)
        return 0

    jax.lax.fori_loop(lo, hi, edge_body, 0)

    @pl.when(j == NCHUNK - 1)
    def _finish_iteration():
        for r in range(NT):
            sl = pl.ds(r * TILE, TILE)
            fr = forest_ref[sl, :]
            fh0 = fh_s[0, sl, :]
            fh1 = fh_s[1, sl, :]
            fh2 = fh_s[2, sl, :]
            i = jax.nn.sigmoid(mm(fr, wiou_t_ref[0]) + wiou_b_ref[0] + mm(fh0, uiou_t_ref[0, 0]) + mm(fh1, uiou_t_ref[0, 1]) + mm(fh2, uiou_t_ref[0, 2]))
            o = jax.nn.sigmoid(mm(fr, wiou_t_ref[1]) + wiou_b_ref[1] + mm(fh0, uiou_t_ref[1, 0]) + mm(fh1, uiou_t_ref[1, 1]) + mm(fh2, uiou_t_ref[1, 2]))
            u = jnp.tanh(mm(fr, wiou_t_ref[2]) + wiou_b_ref[2] + mm(fh0, uiou_t_ref[2, 0]) + mm(fh1, uiou_t_ref[2, 1]) + mm(fh2, uiou_t_ref[2, 2]))
            c_red = wc_b_ref[...] + mm(fc_s[0, sl, :], wc_t_ref[0]) + mm(fc_s[1, sl, :], wc_t_ref[1]) + mm(fc_s[2, sl, :], wc_t_ref[2])
            new_c = i * u + c_red
            new_h = o * jnp.tanh(new_c)
            m = nord_ref[sl, :] == t
            c_s[sl, :] = jnp.where(m, new_c, c_s[sl, :])
            h_out_ref[sl, :] = jnp.where(m, new_h, h_out_ref[sl, :])


def kernel(forest, adjacency, node_order, edge_order, W_iou_w, W_iou_b, U_iou_w, W_c_w, W_c_b, W_f_w, W_f_b, U_f_w):
    f32 = jnp.float32
    forest = forest.reshape(-1, F).astype(f32)
    adj = adjacency[..., :3].reshape(-1, 3)
    parent = adj[:, 0].astype(jnp.int32)
    child = adj[:, 1].astype(jnp.int32)
    slot = jnp.clip(adj[:, 2].astype(jnp.int32) + 1, 0, BF - 1)
    eo = edge_order.reshape(-1).astype(jnp.int32)
    nord = node_order.reshape(-1, 1).astype(jnp.int32)

    valid = (parent >= 0) & (parent < N) & (child >= 0) & (child < N)
    parent = jnp.clip(parent, 0, N - 1)
    child = jnp.clip(child, 0, N - 1)
    key = jnp.where(valid & (eo >= 0) & (eo < MAX_IT), eo, MAX_IT)
    order = jnp.argsort(key)
    key_sorted = key[order]
    packed = ((parent << 16) | (slot << 14) | child)[order]
    packed = jnp.concatenate([packed, jnp.zeros((EPAD - E,), jnp.int32)])
    offs = jnp.searchsorted(key_sorted, jnp.arange(MAX_IT + 1, dtype=jnp.int32)).astype(jnp.int32)

    wf_t = W_f_w.astype(f32).T
    wf_b = W_f_b.astype(f32).reshape(1, F)
    wiou_t = jnp.stack([W_iou_w[j * F:(j + 1) * F, :].astype(f32).T for j in range(3)])
    wiou_b = W_iou_b.astype(f32).reshape(3, 1, F)
    uiou_t = jnp.stack([
        jnp.stack([U_iou_w[j * F:(j + 1) * F, s * F:(s + 1) * F].astype(f32).T for s in range(3)])
        for j in range(3)])
    uf_t = U_f_w.astype(f32).T
    wc_t = jnp.stack([W_c_w[:, s * F:(s + 1) * F].astype(f32).T for s in range(3)])
    wc_b = W_c_b.astype(f32).reshape(1, F)

    full_vmem = pl.BlockSpec(memory_space=pltpu.VMEM)

    return pl.pallas_call(
        _body,
        grid=(MAX_IT, NCHUNK),
        out_shape=jax.ShapeDtypeStruct((N, F), f32),
        in_specs=[
            pl.BlockSpec(memory_space=pltpu.SMEM),
            pl.BlockSpec((C,), lambda t, j: (j,), memory_space=pltpu.SMEM),
            full_vmem,
            full_vmem,
            full_vmem,
            full_vmem,
            full_vmem,
            full_vmem,
            full_vmem,
            full_vmem,
            full_vmem,
            full_vmem,
        ],
        out_specs=full_vmem,
        scratch_shapes=[
            pltpu.VMEM((N, F), f32),
            pltpu.VMEM((N, F), f32),
            pltpu.VMEM((N, F), f32),
            pltpu.VMEM((BF, N, F), f32),
            pltpu.VMEM((BF, N, F), f32),
        ],
        compiler_params=pltpu.CompilerParams(
            dimension_semantics=("arbitrary", "arbitrary"),
            vmem_limit_bytes=65536 * 1024,
        ),
    )(offs, packed, forest, nord, wf_t, wf_b, wiou_t, wiou_b, uiou_t, uf_t, wc_t, wc_b)
